# baseline (speedup 1.0000x reference)
import functools

import jax
import jax.numpy as jnp
from jax.experimental import pallas as pl
from jax.experimental.pallas import tpu as pltpu

_N, _E, _NS, _ES, _S, _G = 10000, 160000, 50000, 200000, 2048, 64
_D, _DE, _ENC, _KMAX, _NOUT = 128, 16, 16, 2, 1
_EPS = 1e-05


def _mm_body(x_ref, w_ref, b_ref, o_ref, *, act):
    y = jnp.dot(x_ref[...], w_ref[...], preferred_element_type=jnp.float32)
    y = y + b_ref[...]
    if act == "relu":
        y = jnp.maximum(y, 0.0)
    o_ref[...] = y


def _tc_mm(x, w, b=None, act=None, block=400):
    m, k = x.shape
    n = w.shape[1]
    assert m % block == 0, (m, block)
    if b is None:
        b = jnp.zeros((n,), jnp.float32)
    b2 = b.reshape(1, n)
    return pl.pallas_call(
        functools.partial(_mm_body, act=act),
        grid=(m // block,),
        in_specs=[
            pl.BlockSpec((block, k), lambda i: (i, 0)),
            pl.BlockSpec((k, n), lambda i: (0, 0)),
            pl.BlockSpec((1, n), lambda i: (0, 0)),
        ],
        out_specs=pl.BlockSpec((block, n), lambda i: (i, 0)),
        out_shape=jax.ShapeDtypeStruct((m, n), jnp.float32),
    )(x, w, b2)


def _stats_body(x_ref, sum_ref, sq_ref):
    i = pl.program_id(0)

    @pl.when(i == 0)
    def _init():
        sum_ref[...] = jnp.zeros_like(sum_ref)
        sq_ref[...] = jnp.zeros_like(sq_ref)

    x = x_ref[...]
    sum_ref[...] += jnp.sum(x, axis=0, keepdims=True)
    sq_ref[...] += jnp.sum(x * x, axis=0, keepdims=True)


def _tc_colstats(x, block=400):
    m, n = x.shape
    assert m % block == 0
    return pl.pallas_call(
        _stats_body,
        grid=(m // block,),
        in_specs=[pl.BlockSpec((block, n), lambda i: (i, 0))],
        out_specs=[pl.BlockSpec((1, n), lambda i: (0, 0)),
                   pl.BlockSpec((1, n), lambda i: (0, 0))],
        out_shape=[jax.ShapeDtypeStruct((1, n), jnp.float32),
                   jax.ShapeDtypeStruct((1, n), jnp.float32)],
    )(x)


def _norm_res_body(y_ref, m_ref, v_ref, prev_ref, o_ref):
    y = y_ref[...]
    hh = (y - m_ref[...]) * jax.lax.rsqrt(v_ref[...] + _EPS)
    o_ref[...] = jnp.maximum(hh, 0.0) + prev_ref[...]


def _tc_norm_residual(y, mean, var, prev, block=400):
    m, n = y.shape
    assert m % block == 0
    return pl.pallas_call(
        _norm_res_body,
        grid=(m // block,),
        in_specs=[
            pl.BlockSpec((block, n), lambda i: (i, 0)),
            pl.BlockSpec((1, n), lambda i: (0, 0)),
            pl.BlockSpec((1, n), lambda i: (0, 0)),
            pl.BlockSpec((block, n), lambda i: (i, 0)),
        ],
        out_specs=pl.BlockSpec((block, n), lambda i: (i, 0)),
        out_shape=jax.ShapeDtypeStruct((m, n), jnp.float32),
    )(y, mean, var, prev)


def kernel(x, edge_attr, W_in, b_in, W_edge, b_edge, W_msg1, W_self1,
           W_msg2, W_self2, W_oe, b_oe, ks_emb, comp_emb, W_sub, b_sub,
           W_bip, b_bip, ks_emb16, comp_emb16, W_gate, W_dec0, b_dec0,
           W_d1, b_d1, W_d2, b_d2, edge_index, subgraphs_nodes_mapper,
           combined_subgraphs, subgraphs_edges_mapper, subgraphs_batch,
           graph_id, ks, num_components):
    src = combined_subgraphs[0]
    dst = combined_subgraphs[1]

    h = _tc_mm(x, W_in, b_in, act="relu")
    ea = _tc_mm(edge_attr, W_edge, b_edge, act="relu")

    prev = jnp.take(h, subgraphs_nodes_mapper, axis=0)

    for Wm, Ws in ((W_msg1, W_self1), (W_msg2, W_self2)):
        p_t = _tc_mm(prev, Wm)
        ea_t = _tc_mm(ea, Wm)
        msg = jnp.maximum(
            jnp.take(p_t, src, axis=0) + jnp.take(ea_t, subgraphs_edges_mapper, axis=0),
            0.0)
        agg = jax.ops.segment_sum(msg, dst, num_segments=_NS)
        y = _tc_mm(agg, Ws)
        s, sq = _tc_colstats(y)
        mean = s / _NS
        var = sq / _NS - mean * mean
        prev = _tc_norm_residual(y, mean, var, prev)

    hsub = _tc_mm(prev, W_oe, b_oe, act="relu")
    subg = jax.ops.segment_sum(hsub, subgraphs_batch, num_segments=_S)

    full = subg / (ks + 1).astype(jnp.float32)[:, None]
    subg = full + jnp.take(ks_emb, ks, axis=0) + jnp.take(comp_emb, num_components, axis=0)
    subg = jax.nn.relu(subg @ W_sub + b_sub)
    subg = jax.nn.relu(subg @ W_bip + b_bip)
    kbatch = ks + graph_id * _KMAX
    kc = jnp.concatenate([jnp.take(ks_emb16, ks, axis=0),
                          jnp.take(comp_emb16, num_components, axis=0)], axis=-1)
    gate = jax.nn.sigmoid(kc @ W_gate)
    max_info = jax.ops.segment_max(subg, kbatch, num_segments=_G * _KMAX)
    max_info = jnp.where(jnp.isfinite(max_info), max_info, 0.0)
    sum_info = jax.ops.segment_sum(subg * gate, kbatch, num_segments=_G * _KMAX)
    xg = jax.nn.relu(jnp.concatenate([max_info, sum_info], axis=-1) @ W_dec0 + b_dec0)
    xg = xg.reshape(_G, _KMAX * _D)
    return jax.nn.relu(xg @ W_d1 + b_d1) @ W_d2 + b_d2


# speedup vs baseline: 1.1874x; 1.1874x over previous
import functools

import jax
import jax.numpy as jnp
from jax import lax
from jax.experimental import pallas as pl
from jax.experimental.pallas import tpu as pltpu
from jax.experimental.pallas import tpu_sc as plsc

_N, _E, _NS, _ES, _S, _G = 10000, 160000, 50000, 200000, 2048, 64
_D, _DE, _ENC, _KMAX, _NOUT = 128, 16, 16, 2, 1
_EPS = 1e-05

_NSP = 51200
_ESP = 204800
_EB = _ESP // 128
_SINK = _NS
_NC = 4
_CW = 32
_SP = 2176
_SSINK = _S


def _mm_body(x_ref, w_ref, b_ref, o_ref, *, act):
    y = jnp.dot(x_ref[...], w_ref[...], preferred_element_type=jnp.float32)
    y = y + b_ref[...]
    if act == "relu":
        y = jnp.maximum(y, 0.0)
    o_ref[...] = y


def _tc_mm(x, w, b=None, act=None, block=400):
    m, k = x.shape
    n = w.shape[1]
    assert m % block == 0, (m, block)
    if b is None:
        b = jnp.zeros((n,), jnp.float32)
    b2 = b.reshape(1, n)
    return pl.pallas_call(
        functools.partial(_mm_body, act=act),
        grid=(m // block,),
        in_specs=[
            pl.BlockSpec((block, k), lambda i: (i, 0)),
            pl.BlockSpec((k, n), lambda i: (0, 0)),
            pl.BlockSpec((1, n), lambda i: (0, 0)),
        ],
        out_specs=pl.BlockSpec((block, n), lambda i: (i, 0)),
        out_shape=jax.ShapeDtypeStruct((m, n), jnp.float32),
    )(x, w, b2)


def _mmc_body(x_ref, w_ref, o_ref):
    o_ref[0] = jnp.dot(x_ref[...], w_ref[0], preferred_element_type=jnp.float32)


def _tc_mm_chunked(x, w, block=400):
    m, k = x.shape
    assert m % block == 0
    w4 = w.reshape(k, _NC, _CW).transpose(1, 0, 2)
    return pl.pallas_call(
        _mmc_body,
        grid=(_NC, m // block),
        in_specs=[
            pl.BlockSpec((block, k), lambda c, i: (i, 0)),
            pl.BlockSpec((1, k, _CW), lambda c, i: (c, 0, 0)),
        ],
        out_specs=pl.BlockSpec((1, block, _CW), lambda c, i: (c, i, 0)),
        out_shape=jax.ShapeDtypeStruct((_NC, m, _CW), jnp.float32),
    )(x, w4)


def _mmfc_body(x_ref, w_ref, o_ref):
    acc = jnp.zeros(o_ref.shape, jnp.float32)
    for c in range(_NC):
        acc += jnp.dot(x_ref[c], w_ref[c], preferred_element_type=jnp.float32)
    o_ref[...] = acc


def _tc_mm_from_chunks(x4, w, block=400):
    _, m, _ = x4.shape
    n = w.shape[1]
    assert m % block == 0
    w4 = w.reshape(_NC, _CW, n)
    return pl.pallas_call(
        _mmfc_body,
        grid=(m // block,),
        in_specs=[
            pl.BlockSpec((_NC, block, _CW), lambda i: (0, i, 0)),
            pl.BlockSpec((_NC, _CW, n), lambda i: (0, 0, 0)),
        ],
        out_specs=pl.BlockSpec((block, n), lambda i: (i, 0)),
        out_shape=jax.ShapeDtypeStruct((m, n), jnp.float32),
    )(x4, w4)


def _stats_body(x_ref, sum_ref, sq_ref):
    i = pl.program_id(0)

    @pl.when(i == 0)
    def _init():
        sum_ref[...] = jnp.zeros_like(sum_ref)
        sq_ref[...] = jnp.zeros_like(sq_ref)

    x = x_ref[...]
    sum_ref[...] += jnp.sum(x, axis=0, keepdims=True)
    sq_ref[...] += jnp.sum(x * x, axis=0, keepdims=True)


def _tc_colstats(x, rows, block=400):
    m, n = x.shape
    assert rows % block == 0
    return pl.pallas_call(
        _stats_body,
        grid=(rows // block,),
        in_specs=[pl.BlockSpec((block, n), lambda i: (i, 0))],
        out_specs=[pl.BlockSpec((1, n), lambda i: (0, 0)),
                   pl.BlockSpec((1, n), lambda i: (0, 0))],
        out_shape=[jax.ShapeDtypeStruct((1, n), jnp.float32),
                   jax.ShapeDtypeStruct((1, n), jnp.float32)],
    )(x)


def _norm_res_body(y_ref, m_ref, v_ref, prev_ref, o_ref):
    y = y_ref[...]
    hh = (y - m_ref[...]) * lax.rsqrt(v_ref[...] + _EPS)
    o_ref[...] = jnp.maximum(hh, 0.0) + prev_ref[...]


def _tc_norm_residual(y, mean, var, prev, block=400):
    m, n = y.shape
    assert m % block == 0
    return pl.pallas_call(
        _norm_res_body,
        grid=(m // block,),
        in_specs=[
            pl.BlockSpec((block, n), lambda i: (i, 0)),
            pl.BlockSpec((1, n), lambda i: (0, 0)),
            pl.BlockSpec((1, n), lambda i: (0, 0)),
            pl.BlockSpec((block, n), lambda i: (i, 0)),
        ],
        out_specs=pl.BlockSpec((block, n), lambda i: (i, 0)),
        out_shape=jax.ShapeDtypeStruct((m, n), jnp.float32),
    )(y, mean, var, prev)


_MESH = plsc.VectorSubcoreMesh(core_axis_name="c", subcore_axis_name="s")
_SC_PARAMS = pltpu.CompilerParams(use_tc_tiling_on_sc=False)


def _edge_body(pt, eat, src4, em4, dst2, zr, agg,
               sv, ev, dv, p_v, e_v, m_v, zbuf, acc, sem):
    core = lax.axis_index("c")
    sub = lax.axis_index("s")
    pltpu.sync_copy(zr, zbuf)

    def compute_row(r, _):
        for hh in (0, 16):
            a = p_v[r, pl.ds(hh, 16)]
            b = e_v[r, pl.ds(hh, 16)]
            m_v[r, pl.ds(hh, 16)] = jnp.maximum(a + b, 0.0)
        return 0

    for cc in range(2):
        c = core * 2 + cc
        for k in range(8):
            pltpu.sync_copy(zbuf, acc.at[pl.ds(sub * 3200 + k * 400, 400)])
        plsc.subcore_barrier()

        def step(j, _):
            row_d = sub * 100 + j
            row = c * 1600 + row_d
            pltpu.sync_copy(src4.at[pl.ds(row, 1)], sv)
            pltpu.sync_copy(em4.at[pl.ds(row, 1)], ev)
            pltpu.sync_copy(dst2.at[pl.ds(row_d, 1)], dv)
            d1 = pltpu.async_copy(pt.at[sv.at[0]], p_v, sem)
            d2 = pltpu.async_copy(eat.at[ev.at[0]], e_v, sem)
            d1.wait()
            d2.wait()
            lax.fori_loop(0, 128, compute_row, 0)
            pltpu.sync_copy(m_v, acc.at[dv.at[0]], add=True)
            return 0

        lax.fori_loop(0, 100, step, 0)
        plsc.subcore_barrier()
        pltpu.sync_copy(acc.at[pl.ds(sub * 3200, 3200)],
                        agg.at[pl.ds(sub * 3200, 3200), c])
        plsc.subcore_barrier()


def _sc_edge_pass(pt_flat, eat_flat, src4, em4, dst2):
    zr = jnp.zeros((400, _CW), jnp.float32)
    fn = pl.kernel(
        _edge_body,
        out_type=jax.ShapeDtypeStruct((_NSP, _NC, _CW), jnp.float32),
        mesh=_MESH,
        compiler_params=_SC_PARAMS,
        scratch_types=[
            pltpu.VMEM((1, 128), jnp.int32),
            pltpu.VMEM((1, 128), jnp.int32),
            pltpu.VMEM((1, 128), jnp.int32),
            pltpu.VMEM((128, _CW), jnp.float32),
            pltpu.VMEM((128, _CW), jnp.float32),
            pltpu.VMEM((128, _CW), jnp.float32),
            pltpu.VMEM((400, _CW), jnp.float32),
            pltpu.VMEM_SHARED((_NSP, _CW), jnp.float32),
            pltpu.SemaphoreType.DMA,
        ],
    )
    return fn(pt_flat, eat_flat, src4, em4, dst2, zr)


def _gather_body(h, idx2, out, iv, rows_v, sem):
    core = lax.axis_index("c")
    sub = lax.axis_index("s")
    wid = sub * 2 + core

    def step(j, _):
        blk = j * 32 + wid

        @pl.when(blk < 400)
        def _():
            pltpu.sync_copy(idx2.at[pl.ds(blk, 1)], iv)
            pltpu.async_copy(h.at[iv.at[0]], rows_v, sem).wait()
            pltpu.sync_copy(rows_v, out.at[pl.ds(blk * 128, 128)])
        return 0

    lax.fori_loop(0, 13, step, 0)


def _sc_gather(h, idx2):
    fn = pl.kernel(
        _gather_body,
        out_type=jax.ShapeDtypeStruct((_NSP, _D), jnp.float32),
        mesh=_MESH,
        compiler_params=_SC_PARAMS,
        scratch_types=[
            pltpu.VMEM((1, 128), jnp.int32),
            pltpu.VMEM((128, _D), jnp.float32),
            pltpu.SemaphoreType.DMA,
        ],
    )
    return fn(h, idx2)


def _segsum_body(xin, idx2, zr, out, iv, x_v, zbuf, acc, sem):
    core = lax.axis_index("c")
    sub = lax.axis_index("s")
    pltpu.sync_copy(zr, zbuf)
    pltpu.sync_copy(zbuf, acc.at[pl.ds(sub * 136, 136)])
    plsc.subcore_barrier()

    def step(j, _):
        off = j * 16 + sub

        @pl.when(off < 200)
        def _():
            blk = core * 200 + off
            pltpu.sync_copy(idx2.at[pl.ds(blk, 1)], iv)
            pltpu.sync_copy(xin.at[pl.ds(blk * 128, 128)], x_v)
            pltpu.sync_copy(x_v, acc.at[iv.at[0]], add=True)
        return 0

    lax.fori_loop(0, 13, step, 0)
    plsc.subcore_barrier()
    pltpu.sync_copy(acc.at[pl.ds(sub * 136, 136)],
                    out.at[core, pl.ds(sub * 136, 136)])


def _sc_segsum(xin, idx2):
    zr = jnp.zeros((136, _D), jnp.float32)
    fn = pl.kernel(
        _segsum_body,
        out_type=jax.ShapeDtypeStruct((2, _SP, _D), jnp.float32),
        mesh=_MESH,
        compiler_params=_SC_PARAMS,
        scratch_types=[
            pltpu.VMEM((1, 128), jnp.int32),
            pltpu.VMEM((128, _D), jnp.float32),
            pltpu.VMEM((136, _D), jnp.float32),
            pltpu.VMEM_SHARED((_SP, _D), jnp.float32),
            pltpu.SemaphoreType.DMA,
        ],
    )
    return fn(xin, idx2, zr)


def kernel(x, edge_attr, W_in, b_in, W_edge, b_edge, W_msg1, W_self1,
           W_msg2, W_self2, W_oe, b_oe, ks_emb, comp_emb, W_sub, b_sub,
           W_bip, b_bip, ks_emb16, comp_emb16, W_gate, W_dec0, b_dec0,
           W_d1, b_d1, W_d2, b_d2, edge_index, subgraphs_nodes_mapper,
           combined_subgraphs, subgraphs_edges_mapper, subgraphs_batch,
           graph_id, ks, num_components):
    src = combined_subgraphs[0]
    dst = combined_subgraphs[1]

    pad_e = _ESP - _ES
    srcp = jnp.concatenate([src, jnp.zeros((pad_e,), jnp.int32)])
    emp = jnp.concatenate([subgraphs_edges_mapper, jnp.zeros((pad_e,), jnp.int32)])
    dstp = jnp.concatenate([dst, jnp.full((pad_e,), _SINK, jnp.int32)])
    offs = jnp.arange(_NC, dtype=jnp.int32)[:, None]
    src4 = (srcp[None, :] * _NC + offs).reshape(_NC * _EB, 128)
    em4 = (emp[None, :] * _NC + offs).reshape(_NC * _EB, 128)
    dst2 = dstp.reshape(_EB, 128)

    mapper_p = jnp.concatenate(
        [subgraphs_nodes_mapper, jnp.zeros((_NSP - _NS,), jnp.int32)]).reshape(400, 128)
    batch_p = jnp.concatenate(
        [subgraphs_batch, jnp.full((_NSP - _NS,), _SSINK, jnp.int32)]).reshape(400, 128)

    h = _tc_mm(x, W_in, b_in, act="relu")
    ea = _tc_mm(edge_attr, W_edge, b_edge, act="relu")

    prev = _sc_gather(h, mapper_p)

    for Wm, Ws in ((W_msg1, W_self1), (W_msg2, W_self2)):
        p_t = _tc_mm(prev, Wm).reshape(_NSP * _NC, _CW)
        ea_t = _tc_mm(ea, Wm).reshape(_E * _NC, _CW)
        agg = _sc_edge_pass(p_t, ea_t, src4, em4, dst2).reshape(_NSP, _D)
        y = _tc_mm(agg, Ws)
        s, sq = _tc_colstats(y, _NS)
        mean = s / _NS
        var = sq / _NS - mean * mean
        prev = _tc_norm_residual(y, mean, var, prev)

    hsub = _tc_mm(prev, W_oe, b_oe, act="relu")
    parts = _sc_segsum(hsub, batch_p)
    subg = (parts[0] + parts[1])[:_S]

    full = subg / (ks + 1).astype(jnp.float32)[:, None]
    subg = full + jnp.take(ks_emb, ks, axis=0) + jnp.take(comp_emb, num_components, axis=0)
    subg = jax.nn.relu(subg @ W_sub + b_sub)
    subg = jax.nn.relu(subg @ W_bip + b_bip)
    kbatch = ks + graph_id * _KMAX
    kc = jnp.concatenate([jnp.take(ks_emb16, ks, axis=0),
                          jnp.take(comp_emb16, num_components, axis=0)], axis=-1)
    gate = jax.nn.sigmoid(kc @ W_gate)
    max_info = jax.ops.segment_max(subg, kbatch, num_segments=_G * _KMAX)
    max_info = jnp.where(jnp.isfinite(max_info), max_info, 0.0)
    sum_info = jax.ops.segment_sum(subg * gate, kbatch, num_segments=_G * _KMAX)
    xg = jax.nn.relu(jnp.concatenate([max_info, sum_info], axis=-1) @ W_dec0 + b_dec0)
    xg = xg.reshape(_G, _KMAX * _D)
    return jax.nn.relu(xg @ W_d1 + b_d1) @ W_d2 + b_d2


# speedup vs baseline: 1.9981x; 1.6827x over previous
import functools

import jax
import jax.numpy as jnp
from jax import lax
from jax.experimental import pallas as pl
from jax.experimental.pallas import tpu as pltpu
from jax.experimental.pallas import tpu_sc as plsc

_N, _E, _NS, _ES, _S, _G = 10000, 160000, 50000, 200000, 2048, 64
_D, _DE, _ENC, _KMAX, _NOUT = 128, 16, 16, 2, 1
_EPS = 1e-05

_NSP = 50048
_NBLK = _NSP // 128
_ESP = 204800
_EB = _ESP // 128
_SINK = _NS
_NC = 4
_CW = 32
_SP = 2176
_SSINK = _S


def _mm_body(x_ref, w_ref, b_ref, o_ref, *, act):
    y = jnp.dot(x_ref[...], w_ref[...], preferred_element_type=jnp.float32)
    y = y + b_ref[...]
    if act == "relu":
        y = jnp.maximum(y, 0.0)
    o_ref[...] = y


def _tc_mm(x, w, b=None, act=None, block=400):
    m, k = x.shape
    n = w.shape[1]
    assert m % block == 0, (m, block)
    if b is None:
        b = jnp.zeros((n,), jnp.float32)
    b2 = b.reshape(1, n)
    return pl.pallas_call(
        functools.partial(_mm_body, act=act),
        grid=(m // block,),
        in_specs=[
            pl.BlockSpec((block, k), lambda i: (i, 0)),
            pl.BlockSpec((k, n), lambda i: (0, 0)),
            pl.BlockSpec((1, n), lambda i: (0, 0)),
        ],
        out_specs=pl.BlockSpec((block, n), lambda i: (i, 0)),
        out_shape=jax.ShapeDtypeStruct((m, n), jnp.float32),
    )(x, w, b2)


def _stats_body(x_ref, sum_ref, sq_ref):
    i = pl.program_id(0)

    @pl.when(i == 0)
    def _init():
        sum_ref[...] = jnp.zeros_like(sum_ref)
        sq_ref[...] = jnp.zeros_like(sq_ref)

    x = x_ref[...]
    sum_ref[...] += jnp.sum(x, axis=0, keepdims=True)
    sq_ref[...] += jnp.sum(x * x, axis=0, keepdims=True)


def _tc_colstats(x, rows, block=400):
    m, n = x.shape
    assert rows % block == 0
    return pl.pallas_call(
        _stats_body,
        grid=(rows // block,),
        in_specs=[pl.BlockSpec((block, n), lambda i: (i, 0))],
        out_specs=[pl.BlockSpec((1, n), lambda i: (0, 0)),
                   pl.BlockSpec((1, n), lambda i: (0, 0))],
        out_shape=[jax.ShapeDtypeStruct((1, n), jnp.float32),
                   jax.ShapeDtypeStruct((1, n), jnp.float32)],
    )(x)


def _norm_res_body(y_ref, m_ref, v_ref, prev_ref, o_ref):
    y = y_ref[...]
    hh = (y - m_ref[...]) * lax.rsqrt(v_ref[...] + _EPS)
    o_ref[...] = jnp.maximum(hh, 0.0) + prev_ref[...]


def _tc_norm_residual(y, mean, var, prev, block=1088):
    m, n = y.shape
    assert m % block == 0
    return pl.pallas_call(
        _norm_res_body,
        grid=(m // block,),
        in_specs=[
            pl.BlockSpec((block, n), lambda i: (i, 0)),
            pl.BlockSpec((1, n), lambda i: (0, 0)),
            pl.BlockSpec((1, n), lambda i: (0, 0)),
            pl.BlockSpec((block, n), lambda i: (i, 0)),
        ],
        out_specs=pl.BlockSpec((block, n), lambda i: (i, 0)),
        out_shape=jax.ShapeDtypeStruct((m, n), jnp.float32),
    )(y, mean, var, prev)


_MESH = plsc.VectorSubcoreMesh(core_axis_name="c", subcore_axis_name="s")
_SC_PARAMS = pltpu.CompilerParams(use_tc_tiling_on_sc=False)


def _edge_body(pt, eat, src4, em4, dst2, zr, agg,
               sv, ev, dv, p_v, e_v, m_v, acc,
               g0, g1, s0, s1, i0, i1):
    core = lax.axis_index("c")
    sub = lax.axis_index("s")
    sem_g = (g0, g1)
    sem_s = (s0, s1)
    sem_i = (i0, i1)

    def issue_gather(bank, k, b):
        pltpu.async_copy(pt.at[sv.at[bank, k]], p_v.at[b], sem_g[b])
        pltpu.async_copy(eat.at[ev.at[bank, k]], e_v.at[b], sem_g[b])

    def wait_gather(bank, k, b):
        pltpu.make_async_copy(pt.at[sv.at[bank, k]], p_v.at[b], sem_g[b]).wait()
        pltpu.make_async_copy(eat.at[ev.at[bank, k]], e_v.at[b], sem_g[b]).wait()

    def issue_scatter(bank, k, b):
        pltpu.async_copy(m_v.at[b], acc.at[dv.at[bank, k]], sem_s[b], add=True)

    def wait_scatter(b):
        pltpu.make_async_copy(m_v.at[b], acc.at[dv.at[0, 0]], sem_s[b]).wait()

    def load_idx_sync(t, bank, base):
        pltpu.sync_copy(src4.at[pl.ds(base + 5 * t, 5)], sv.at[bank])
        pltpu.sync_copy(em4.at[pl.ds(base + 5 * t, 5)], ev.at[bank])
        pltpu.sync_copy(dst2.at[pl.ds(sub * 100 + 5 * t, 5)], dv.at[bank])

    def load_idx_async(t, bank, base):
        pltpu.async_copy(src4.at[pl.ds(base + 5 * t, 5)], sv.at[bank], sem_i[bank])
        pltpu.async_copy(em4.at[pl.ds(base + 5 * t, 5)], ev.at[bank], sem_i[bank])
        pltpu.async_copy(dst2.at[pl.ds(sub * 100 + 5 * t, 5)], dv.at[bank], sem_i[bank])

    def wait_idx(bank, base):
        pltpu.make_async_copy(src4.at[pl.ds(base, 5)], sv.at[bank], sem_i[bank]).wait()
        pltpu.make_async_copy(em4.at[pl.ds(base, 5)], ev.at[bank], sem_i[bank]).wait()
        pltpu.make_async_copy(dst2.at[pl.ds(base, 5)], dv.at[bank], sem_i[bank]).wait()

    def compute(b):
        def comp_iter(i, _):
            for rr in range(4):
                r = i * 4 + rr
                for hh in (0, 16):
                    a = p_v[b, r, pl.ds(hh, 16)]
                    bb = e_v[b, r, pl.ds(hh, 16)]
                    m_v[b, r, pl.ds(hh, 16)] = jnp.maximum(a + bb, 0.0)
            return 0
        lax.fori_loop(0, 32, comp_iter, 0)

    for cc in range(2):
        c = core * 2 + cc
        base = c * 1600 + sub * 100
        pltpu.sync_copy(zr, acc.at[pl.ds(sub * 3128, 3128)])
        plsc.subcore_barrier()

        load_idx_sync(0, 0, base)
        issue_gather(0, 0, 0)

        def period(t, tp):
            bt = tp
            nb = 1 - tp

            @pl.when(t > 0)
            def _():
                wait_scatter(tp)
                wait_scatter(1 - tp)

            @pl.when(t < 19)
            def _():
                load_idx_async(t + 1, nb, base)

            for k in range(5):
                pb = (tp + k) % 2
                if k < 4:
                    issue_gather(bt, k + 1, 1 - pb)
                else:
                    @pl.when(t < 19)
                    def _():
                        wait_idx(nb, base)
                        issue_gather(nb, 0, 1 - pb)
                wait_gather(bt, k, pb)
                if k >= 2:
                    wait_scatter(pb)
                compute(pb)
                issue_scatter(bt, k, pb)

        def two(s, _):
            period(2 * s, 0)
            period(2 * s + 1, 1)
            return 0

        lax.fori_loop(0, 10, two, 0)
        wait_scatter(0)
        wait_scatter(1)
        plsc.subcore_barrier()
        pltpu.sync_copy(acc.at[pl.ds(sub * 3128, 3128)],
                        agg.at[pl.ds(sub * 3128, 3128), c])
        plsc.subcore_barrier()


def _sc_edge_pass(pt_flat, eat_flat, src4, em4, dst2):
    zr = jnp.zeros((3128, _CW), jnp.float32)
    fn = pl.kernel(
        _edge_body,
        out_type=jax.ShapeDtypeStruct((_NSP, _NC, _CW), jnp.float32),
        mesh=_MESH,
        compiler_params=_SC_PARAMS,
        scratch_types=[
            pltpu.VMEM((2, 5, 128), jnp.int32),
            pltpu.VMEM((2, 5, 128), jnp.int32),
            pltpu.VMEM((2, 5, 128), jnp.int32),
            pltpu.VMEM((2, 128, _CW), jnp.float32),
            pltpu.VMEM((2, 128, _CW), jnp.float32),
            pltpu.VMEM((2, 128, _CW), jnp.float32),
            pltpu.VMEM_SHARED((_NSP, _CW), jnp.float32),
            pltpu.SemaphoreType.DMA,
            pltpu.SemaphoreType.DMA,
            pltpu.SemaphoreType.DMA,
            pltpu.SemaphoreType.DMA,
            pltpu.SemaphoreType.DMA,
            pltpu.SemaphoreType.DMA,
        ],
    )
    return fn(pt_flat, eat_flat, src4, em4, dst2, zr)


def _gather_body(h, idx2, out, iv, rows_v, s0, s1):
    core = lax.axis_index("c")
    sub = lax.axis_index("s")
    wid = sub * 2 + core
    sems = (s0, s1)

    def issue(j, b):
        pltpu.sync_copy(idx2.at[j * 32 + wid], iv.at[b])
        pltpu.async_copy(h.at[iv.at[b]], rows_v.at[b], sems[b])

    issue(0, 0)
    for j in range(13):
        b = j % 2
        if j + 1 < 13:
            @pl.when((j + 1) * 32 + wid < _NBLK)
            def _():
                issue(j + 1, 1 - b)

        @pl.when(j * 32 + wid < _NBLK)
        def _():
            pltpu.make_async_copy(h.at[iv.at[b]], rows_v.at[b], sems[b]).wait()
            pltpu.sync_copy(rows_v.at[b],
                            out.at[pl.ds((j * 32 + wid) * 128, 128)])


def _sc_gather(h, idx2):
    fn = pl.kernel(
        _gather_body,
        out_type=jax.ShapeDtypeStruct((_NSP, _D), jnp.float32),
        mesh=_MESH,
        compiler_params=_SC_PARAMS,
        scratch_types=[
            pltpu.VMEM((2, 128), jnp.int32),
            pltpu.VMEM((2, 128, _D), jnp.float32),
            pltpu.SemaphoreType.DMA,
            pltpu.SemaphoreType.DMA,
        ],
    )
    return fn(h, idx2)


def _segsum_body(xin, idx2, zr, out, iv, x_v, acc, sem):
    core = lax.axis_index("c")
    sub = lax.axis_index("s")
    pltpu.sync_copy(zr, acc.at[pl.ds(sub * 136, 136)])
    plsc.subcore_barrier()

    def step(j, _):
        off = j * 16 + sub

        @pl.when(off < 196 - core)
        def _():
            blk = core * 196 + off
            pltpu.sync_copy(idx2.at[pl.ds(blk, 1)], iv)
            pltpu.sync_copy(xin.at[pl.ds(blk * 128, 128)], x_v)
            pltpu.sync_copy(x_v, acc.at[iv.at[0]], add=True)
        return 0

    lax.fori_loop(0, 13, step, 0)
    plsc.subcore_barrier()
    pltpu.sync_copy(acc.at[pl.ds(sub * 136, 136)],
                    out.at[core, pl.ds(sub * 136, 136)])


def _sc_segsum(xin, idx2):
    zr = jnp.zeros((136, _D), jnp.float32)
    fn = pl.kernel(
        _segsum_body,
        out_type=jax.ShapeDtypeStruct((2, _SP, _D), jnp.float32),
        mesh=_MESH,
        compiler_params=_SC_PARAMS,
        scratch_types=[
            pltpu.VMEM((1, 128), jnp.int32),
            pltpu.VMEM((128, _D), jnp.float32),
            pltpu.VMEM_SHARED((_SP, _D), jnp.float32),
            pltpu.SemaphoreType.DMA,
        ],
    )
    return fn(xin, idx2, zr)


def kernel(x, edge_attr, W_in, b_in, W_edge, b_edge, W_msg1, W_self1,
           W_msg2, W_self2, W_oe, b_oe, ks_emb, comp_emb, W_sub, b_sub,
           W_bip, b_bip, ks_emb16, comp_emb16, W_gate, W_dec0, b_dec0,
           W_d1, b_d1, W_d2, b_d2, edge_index, subgraphs_nodes_mapper,
           combined_subgraphs, subgraphs_edges_mapper, subgraphs_batch,
           graph_id, ks, num_components):
    src = combined_subgraphs[0]
    dst = combined_subgraphs[1]

    pad_e = _ESP - _ES
    srcp = jnp.concatenate([src, jnp.zeros((pad_e,), jnp.int32)])
    emp = jnp.concatenate([subgraphs_edges_mapper, jnp.zeros((pad_e,), jnp.int32)])
    dstp = jnp.concatenate([dst, jnp.full((pad_e,), _SINK, jnp.int32)])
    offs = jnp.arange(_NC, dtype=jnp.int32)[:, None]
    src4 = (srcp[None, :] * _NC + offs).reshape(_NC * _EB, 128)
    em4 = (emp[None, :] * _NC + offs).reshape(_NC * _EB, 128)
    dst2 = dstp.reshape(_EB, 128)

    mapper_p = jnp.concatenate(
        [subgraphs_nodes_mapper, jnp.zeros((_NSP - _NS,), jnp.int32)]).reshape(_NBLK, 128)
    batch_p = jnp.concatenate(
        [subgraphs_batch, jnp.full((_NSP - _NS,), _SSINK, jnp.int32)]).reshape(_NBLK, 128)

    h = _tc_mm(x, W_in, b_in, act="relu", block=400)
    ea = _tc_mm(edge_attr, W_edge, b_edge, act="relu", block=800)

    prev = _sc_gather(h, mapper_p)

    for Wm, Ws in ((W_msg1, W_self1), (W_msg2, W_self2)):
        p_t = _tc_mm(prev, Wm, block=1088).reshape(_NSP * _NC, _CW)
        ea_t = _tc_mm(ea, Wm, block=800).reshape(_E * _NC, _CW)
        agg = _sc_edge_pass(p_t, ea_t, src4, em4, dst2).reshape(_NSP, _D)
        y = _tc_mm(agg, Ws, block=1088)
        s, sq = _tc_colstats(y, _NS, block=400)
        mean = s / _NS
        var = sq / _NS - mean * mean
        prev = _tc_norm_residual(y, mean, var, prev, block=1088)

    hsub = _tc_mm(prev, W_oe, b_oe, act="relu", block=1088)
    parts = _sc_segsum(hsub, batch_p)
    subg = (parts[0] + parts[1])[:_S]

    full = subg / (ks + 1).astype(jnp.float32)[:, None]
    subg = full + jnp.take(ks_emb, ks, axis=0) + jnp.take(comp_emb, num_components, axis=0)
    subg = jax.nn.relu(subg @ W_sub + b_sub)
    subg = jax.nn.relu(subg @ W_bip + b_bip)
    kbatch = ks + graph_id * _KMAX
    kc = jnp.concatenate([jnp.take(ks_emb16, ks, axis=0),
                          jnp.take(comp_emb16, num_components, axis=0)], axis=-1)
    gate = jax.nn.sigmoid(kc @ W_gate)
    max_info = jax.ops.segment_max(subg, kbatch, num_segments=_G * _KMAX)
    max_info = jnp.where(jnp.isfinite(max_info), max_info, 0.0)
    sum_info = jax.ops.segment_sum(subg * gate, kbatch, num_segments=_G * _KMAX)
    xg = jax.nn.relu(jnp.concatenate([max_info, sum_info], axis=-1) @ W_dec0 + b_dec0)
    xg = xg.reshape(_G, _KMAX * _D)
    return jax.nn.relu(xg @ W_d1 + b_d1) @ W_d2 + b_d2


# speedup vs baseline: 2.0059x; 1.0039x over previous
import functools

import jax
import jax.numpy as jnp
from jax import lax
from jax.experimental import pallas as pl
from jax.experimental.pallas import tpu as pltpu
from jax.experimental.pallas import tpu_sc as plsc

_N, _E, _NS, _ES, _S, _G = 10000, 160000, 50000, 200000, 2048, 64
_D, _DE, _ENC, _KMAX, _NOUT = 128, 16, 16, 2, 1
_EPS = 1e-05

_NSP = 50048
_NBLK = _NSP // 128
_ESP = 204800
_EB = _ESP // 128
_SINK = _NS
_NC = 4
_CW = 32
_SP = 2176
_SSINK = _S


def _mm_body(x_ref, w_ref, b_ref, o_ref, *, act):
    y = jnp.dot(x_ref[...], w_ref[...], preferred_element_type=jnp.float32)
    y = y + b_ref[...]
    if act == "relu":
        y = jnp.maximum(y, 0.0)
    o_ref[...] = y


def _tc_mm(x, w, b=None, act=None, block=400):
    m, k = x.shape
    n = w.shape[1]
    assert m % block == 0, (m, block)
    if b is None:
        b = jnp.zeros((n,), jnp.float32)
    b2 = b.reshape(1, n)
    return pl.pallas_call(
        functools.partial(_mm_body, act=act),
        grid=(m // block,),
        in_specs=[
            pl.BlockSpec((block, k), lambda i: (i, 0)),
            pl.BlockSpec((k, n), lambda i: (0, 0)),
            pl.BlockSpec((1, n), lambda i: (0, 0)),
        ],
        out_specs=pl.BlockSpec((block, n), lambda i: (i, 0)),
        out_shape=jax.ShapeDtypeStruct((m, n), jnp.float32),
    )(x, w, b2)


def _stats_body(x_ref, sum_ref, sq_ref):
    i = pl.program_id(0)

    @pl.when(i == 0)
    def _init():
        sum_ref[...] = jnp.zeros_like(sum_ref)
        sq_ref[...] = jnp.zeros_like(sq_ref)

    x = x_ref[...]
    sum_ref[...] += jnp.sum(x, axis=0, keepdims=True)
    sq_ref[...] += jnp.sum(x * x, axis=0, keepdims=True)


def _tc_colstats(x, rows, block=400):
    m, n = x.shape
    assert rows % block == 0
    return pl.pallas_call(
        _stats_body,
        grid=(rows // block,),
        in_specs=[pl.BlockSpec((block, n), lambda i: (i, 0))],
        out_specs=[pl.BlockSpec((1, n), lambda i: (0, 0)),
                   pl.BlockSpec((1, n), lambda i: (0, 0))],
        out_shape=[jax.ShapeDtypeStruct((1, n), jnp.float32),
                   jax.ShapeDtypeStruct((1, n), jnp.float32)],
    )(x)


def _norm_res_body(y_ref, m_ref, v_ref, prev_ref, o_ref):
    y = y_ref[...]
    hh = (y - m_ref[...]) * lax.rsqrt(v_ref[...] + _EPS)
    o_ref[...] = jnp.maximum(hh, 0.0) + prev_ref[...]


def _tc_norm_residual(y, mean, var, prev, block=1088):
    m, n = y.shape
    assert m % block == 0
    return pl.pallas_call(
        _norm_res_body,
        grid=(m // block,),
        in_specs=[
            pl.BlockSpec((block, n), lambda i: (i, 0)),
            pl.BlockSpec((1, n), lambda i: (0, 0)),
            pl.BlockSpec((1, n), lambda i: (0, 0)),
            pl.BlockSpec((block, n), lambda i: (i, 0)),
        ],
        out_specs=pl.BlockSpec((block, n), lambda i: (i, 0)),
        out_shape=jax.ShapeDtypeStruct((m, n), jnp.float32),
    )(y, mean, var, prev)


def _tail_body(parts_ref, ohk_ref, ohc_ref, kbf_ref, ksf_ref,
               kse_ref, cpe_ref, gk_ref, gc_ref,
               wsub_ref, bsub_ref, wbip_ref, bbip_ref,
               wd0a_ref, wd0b_ref, bd0_ref, wd1a_ref, wd1b_ref, bd1_ref,
               wd2_ref, bd2_ref, o_ref, mi_ref):
    f32 = jnp.float32
    subg = (parts_ref[0] + parts_ref[1])[:_S]
    ohk = ohk_ref[...]
    ohc = ohc_ref[...]
    kbf = kbf_ref[...]
    dot = functools.partial(jnp.dot, preferred_element_type=f32)
    subg = subg * ksf_ref[...] + dot(ohk, kse_ref[...]) + dot(ohc, cpe_ref[...])
    subg = jnp.maximum(dot(subg, wsub_ref[...]) + bsub_ref[...], 0.0)
    subg = jnp.maximum(dot(subg, wbip_ref[...]) + bbip_ref[...], 0.0)
    gate = jax.nn.sigmoid(dot(ohk, gk_ref[...]) + dot(ohc, gc_ref[...]))

    def seg_max(g, _):
        mask = kbf == g.astype(f32)
        m = jnp.max(jnp.where(mask, subg, -jnp.inf), axis=0, keepdims=True)
        mi_ref[pl.ds(g, 1), :] = m
        return 0
    lax.fori_loop(0, _G * _KMAX, seg_max, 0)
    mi = mi_ref[...]
    mi = jnp.where(mi == -jnp.inf, 0.0, mi)

    col = lax.broadcasted_iota(jnp.int32, (_S, _G * _KMAX), 1).astype(f32)
    ohkb = jnp.where(kbf == col, 1.0, 0.0)
    si = lax.dot_general(ohkb, subg * gate, (((0,), (0,)), ((), ())),
                         preferred_element_type=f32)

    xg = jnp.maximum(dot(mi, wd0a_ref[...]) + dot(si, wd0b_ref[...])
                     + bd0_ref[...], 0.0)
    xg3 = xg.reshape(_G, _KMAX, _D)
    y1 = jnp.maximum(dot(xg3[:, 0, :], wd1a_ref[...])
                     + dot(xg3[:, 1, :], wd1b_ref[...]) + bd1_ref[...], 0.0)
    o_ref[...] = dot(y1, wd2_ref[...]) + bd2_ref[...]


def _tc_tail(parts, ohk, ohc, kbf, ksf, ks_emb, comp_emb, gk, gc,
             W_sub, b_sub, W_bip, b_bip, wd0a, wd0b, b_dec0,
             wd1a, wd1b, b_d1, W_d2, b_d2):
    args = (parts, ohk, ohc, kbf, ksf, ks_emb, comp_emb, gk, gc,
            W_sub, b_sub.reshape(1, _D), W_bip, b_bip.reshape(1, _D),
            wd0a, wd0b, b_dec0.reshape(1, _D), wd1a, wd1b,
            b_d1.reshape(1, _D), W_d2, b_d2.reshape(1, _NOUT))
    return pl.pallas_call(
        _tail_body,
        grid=(1,),
        in_specs=[pl.BlockSpec(a.shape, lambda i, n=len(a.shape): (0,) * n)
                  for a in args],
        out_specs=pl.BlockSpec((_G, _NOUT), lambda i: (0, 0)),
        out_shape=jax.ShapeDtypeStruct((_G, _NOUT), jnp.float32),
        scratch_shapes=[pltpu.VMEM((_G * _KMAX, _D), jnp.float32)],
    )(*args)


_MESH = plsc.VectorSubcoreMesh(core_axis_name="c", subcore_axis_name="s")
_SC_PARAMS = pltpu.CompilerParams(use_tc_tiling_on_sc=False)


def _edge_body(pt, eat, src4, em4, dst2, zr, agg,
               sv, ev, dv, p_v, e_v, m_v, acc,
               g0, g1, s0, s1, i0, i1):
    core = lax.axis_index("c")
    sub = lax.axis_index("s")
    sem_g = (g0, g1)
    sem_s = (s0, s1)
    sem_i = (i0, i1)

    def issue_gather(bank, k, b):
        pltpu.async_copy(pt.at[sv.at[bank, k]], p_v.at[b], sem_g[b])
        pltpu.async_copy(eat.at[ev.at[bank, k]], e_v.at[b], sem_g[b])

    def wait_gather(bank, k, b):
        pltpu.make_async_copy(pt.at[sv.at[bank, k]], p_v.at[b], sem_g[b]).wait()
        pltpu.make_async_copy(eat.at[ev.at[bank, k]], e_v.at[b], sem_g[b]).wait()

    def issue_scatter(bank, k, b):
        pltpu.async_copy(m_v.at[b], acc.at[dv.at[bank, k]], sem_s[b], add=True)

    def wait_scatter(b):
        pltpu.make_async_copy(m_v.at[b], acc.at[dv.at[0, 0]], sem_s[b]).wait()

    def load_idx_sync(t, bank, base):
        pltpu.sync_copy(src4.at[pl.ds(base + 5 * t, 5)], sv.at[bank])
        pltpu.sync_copy(em4.at[pl.ds(base + 5 * t, 5)], ev.at[bank])
        pltpu.sync_copy(dst2.at[pl.ds(sub * 100 + 5 * t, 5)], dv.at[bank])

    def load_idx_async(t, bank, base):
        pltpu.async_copy(src4.at[pl.ds(base + 5 * t, 5)], sv.at[bank], sem_i[bank])
        pltpu.async_copy(em4.at[pl.ds(base + 5 * t, 5)], ev.at[bank], sem_i[bank])
        pltpu.async_copy(dst2.at[pl.ds(sub * 100 + 5 * t, 5)], dv.at[bank], sem_i[bank])

    def wait_idx(bank, base):
        pltpu.make_async_copy(src4.at[pl.ds(base, 5)], sv.at[bank], sem_i[bank]).wait()
        pltpu.make_async_copy(em4.at[pl.ds(base, 5)], ev.at[bank], sem_i[bank]).wait()
        pltpu.make_async_copy(dst2.at[pl.ds(base, 5)], dv.at[bank], sem_i[bank]).wait()

    def compute(b):
        def comp_iter(i, _):
            for rr in range(4):
                r = i * 4 + rr
                for hh in (0, 16):
                    a = p_v[b, r, pl.ds(hh, 16)]
                    bb = e_v[b, r, pl.ds(hh, 16)]
                    m_v[b, r, pl.ds(hh, 16)] = jnp.maximum(a + bb, 0.0)
            return 0
        lax.fori_loop(0, 32, comp_iter, 0)

    for cc in range(2):
        c = core * 2 + cc
        base = c * 1600 + sub * 100
        pltpu.sync_copy(zr, acc.at[pl.ds(sub * 3128, 3128)])
        plsc.subcore_barrier()

        load_idx_sync(0, 0, base)
        issue_gather(0, 0, 0)

        def period(t, tp):
            bt = tp
            nb = 1 - tp

            @pl.when(t > 0)
            def _():
                wait_scatter(tp)
                wait_scatter(1 - tp)

            @pl.when(t < 19)
            def _():
                load_idx_async(t + 1, nb, base)

            for k in range(5):
                pb = (tp + k) % 2
                if k < 4:
                    issue_gather(bt, k + 1, 1 - pb)
                else:
                    @pl.when(t < 19)
                    def _():
                        wait_idx(nb, base)
                        issue_gather(nb, 0, 1 - pb)
                wait_gather(bt, k, pb)
                if k >= 2:
                    wait_scatter(pb)
                compute(pb)
                issue_scatter(bt, k, pb)

        def two(s, _):
            period(2 * s, 0)
            period(2 * s + 1, 1)
            return 0

        lax.fori_loop(0, 10, two, 0)
        wait_scatter(0)
        wait_scatter(1)
        plsc.subcore_barrier()
        pltpu.sync_copy(acc.at[pl.ds(sub * 3128, 3128)],
                        agg.at[pl.ds(sub * 3128, 3128), c])
        plsc.subcore_barrier()


def _sc_edge_pass(pt_flat, eat_flat, src4, em4, dst2):
    zr = jnp.zeros((3128, _CW), jnp.float32)
    fn = pl.kernel(
        _edge_body,
        out_type=jax.ShapeDtypeStruct((_NSP, _NC, _CW), jnp.float32),
        mesh=_MESH,
        compiler_params=_SC_PARAMS,
        scratch_types=[
            pltpu.VMEM((2, 5, 128), jnp.int32),
            pltpu.VMEM((2, 5, 128), jnp.int32),
            pltpu.VMEM((2, 5, 128), jnp.int32),
            pltpu.VMEM((2, 128, _CW), jnp.float32),
            pltpu.VMEM((2, 128, _CW), jnp.float32),
            pltpu.VMEM((2, 128, _CW), jnp.float32),
            pltpu.VMEM_SHARED((_NSP, _CW), jnp.float32),
            pltpu.SemaphoreType.DMA,
            pltpu.SemaphoreType.DMA,
            pltpu.SemaphoreType.DMA,
            pltpu.SemaphoreType.DMA,
            pltpu.SemaphoreType.DMA,
            pltpu.SemaphoreType.DMA,
        ],
    )
    return fn(pt_flat, eat_flat, src4, em4, dst2, zr)


def _gather_body(h, idx2, out, iv, rows_v, s0, s1):
    core = lax.axis_index("c")
    sub = lax.axis_index("s")
    wid = sub * 2 + core
    sems = (s0, s1)

    def issue(j, b):
        pltpu.sync_copy(idx2.at[j * 32 + wid], iv.at[b])
        pltpu.async_copy(h.at[iv.at[b]], rows_v.at[b], sems[b])

    issue(0, 0)
    for j in range(13):
        b = j % 2
        if j + 1 < 13:
            @pl.when((j + 1) * 32 + wid < _NBLK)
            def _():
                issue(j + 1, 1 - b)

        @pl.when(j * 32 + wid < _NBLK)
        def _():
            pltpu.make_async_copy(h.at[iv.at[b]], rows_v.at[b], sems[b]).wait()
            pltpu.sync_copy(rows_v.at[b],
                            out.at[pl.ds((j * 32 + wid) * 128, 128)])


def _sc_gather(h, idx2):
    fn = pl.kernel(
        _gather_body,
        out_type=jax.ShapeDtypeStruct((_NSP, _D), jnp.float32),
        mesh=_MESH,
        compiler_params=_SC_PARAMS,
        scratch_types=[
            pltpu.VMEM((2, 128), jnp.int32),
            pltpu.VMEM((2, 128, _D), jnp.float32),
            pltpu.SemaphoreType.DMA,
            pltpu.SemaphoreType.DMA,
        ],
    )
    return fn(h, idx2)


def _segsum_body(xin, idx2, zr, out, iv, x_v, acc, sem):
    core = lax.axis_index("c")
    sub = lax.axis_index("s")
    pltpu.sync_copy(zr, acc.at[pl.ds(sub * 136, 136)])
    plsc.subcore_barrier()

    def step(j, _):
        off = j * 16 + sub

        @pl.when(off < 196 - core)
        def _():
            blk = core * 196 + off
            pltpu.sync_copy(idx2.at[pl.ds(blk, 1)], iv)
            pltpu.sync_copy(xin.at[pl.ds(blk * 128, 128)], x_v)
            pltpu.sync_copy(x_v, acc.at[iv.at[0]], add=True)
        return 0

    lax.fori_loop(0, 13, step, 0)
    plsc.subcore_barrier()
    pltpu.sync_copy(acc.at[pl.ds(sub * 136, 136)],
                    out.at[core, pl.ds(sub * 136, 136)])


def _sc_segsum(xin, idx2):
    zr = jnp.zeros((136, _D), jnp.float32)
    fn = pl.kernel(
        _segsum_body,
        out_type=jax.ShapeDtypeStruct((2, _SP, _D), jnp.float32),
        mesh=_MESH,
        compiler_params=_SC_PARAMS,
        scratch_types=[
            pltpu.VMEM((1, 128), jnp.int32),
            pltpu.VMEM((128, _D), jnp.float32),
            pltpu.VMEM_SHARED((_SP, _D), jnp.float32),
            pltpu.SemaphoreType.DMA,
        ],
    )
    return fn(xin, idx2, zr)


def kernel(x, edge_attr, W_in, b_in, W_edge, b_edge, W_msg1, W_self1,
           W_msg2, W_self2, W_oe, b_oe, ks_emb, comp_emb, W_sub, b_sub,
           W_bip, b_bip, ks_emb16, comp_emb16, W_gate, W_dec0, b_dec0,
           W_d1, b_d1, W_d2, b_d2, edge_index, subgraphs_nodes_mapper,
           combined_subgraphs, subgraphs_edges_mapper, subgraphs_batch,
           graph_id, ks, num_components):
    src = combined_subgraphs[0]
    dst = combined_subgraphs[1]

    pad_e = _ESP - _ES
    srcp = jnp.concatenate([src, jnp.zeros((pad_e,), jnp.int32)])
    emp = jnp.concatenate([subgraphs_edges_mapper, jnp.zeros((pad_e,), jnp.int32)])
    dstp = jnp.concatenate([dst, jnp.full((pad_e,), _SINK, jnp.int32)])
    offs = jnp.arange(_NC, dtype=jnp.int32)[:, None]
    src4 = (srcp[None, :] * _NC + offs).reshape(_NC * _EB, 128)
    em4 = (emp[None, :] * _NC + offs).reshape(_NC * _EB, 128)
    dst2 = dstp.reshape(_EB, 128)

    mapper_p = jnp.concatenate(
        [subgraphs_nodes_mapper, jnp.zeros((_NSP - _NS,), jnp.int32)]).reshape(_NBLK, 128)
    batch_p = jnp.concatenate(
        [subgraphs_batch, jnp.full((_NSP - _NS,), _SSINK, jnp.int32)]).reshape(_NBLK, 128)

    h = _tc_mm(x, W_in, b_in, act="relu", block=400)
    ea = _tc_mm(edge_attr, W_edge, b_edge, act="relu", block=800)

    prev = _sc_gather(h, mapper_p)

    for Wm, Ws in ((W_msg1, W_self1), (W_msg2, W_self2)):
        p_t = _tc_mm(prev, Wm, block=1088).reshape(_NSP * _NC, _CW)
        ea_t = _tc_mm(ea, Wm, block=800).reshape(_E * _NC, _CW)
        agg = _sc_edge_pass(p_t, ea_t, src4, em4, dst2).reshape(_NSP, _D)
        y = _tc_mm(agg, Ws, block=1088)
        s, sq = _tc_colstats(y, _NS, block=400)
        mean = s / _NS
        var = sq / _NS - mean * mean
        prev = _tc_norm_residual(y, mean, var, prev, block=1088)

    hsub = _tc_mm(prev, W_oe, b_oe, act="relu", block=1088)
    parts = _sc_segsum(hsub, batch_p)

    lanes = jnp.arange(20, dtype=jnp.int32)
    ohk = (ks[:, None] == lanes).astype(jnp.float32)
    ohc = (num_components[:, None] == lanes).astype(jnp.float32)
    kbf = (ks + graph_id * _KMAX).astype(jnp.float32)[:, None]
    ksf = 1.0 / (ks + 1).astype(jnp.float32)[:, None]
    gk = ks_emb16 @ W_gate[:_ENC]
    gc = comp_emb16 @ W_gate[_ENC:]
    wd0a, wd0b = W_dec0[:_D], W_dec0[_D:]
    wd1a, wd1b = W_d1[:_D], W_d1[_D:]
    return _tc_tail(parts, ohk, ohc, kbf, ksf, ks_emb, comp_emb, gk, gc,
                    W_sub, b_sub, W_bip, b_bip, wd0a, wd0b, b_dec0,
                    wd1a, wd1b, b_d1, W_d2, b_d2)


# speedup vs baseline: 2.1423x; 1.0680x over previous
import functools

import jax
import jax.numpy as jnp
from jax import lax
from jax.experimental import pallas as pl
from jax.experimental.pallas import tpu as pltpu
from jax.experimental.pallas import tpu_sc as plsc

_N, _E, _NS, _ES, _S, _G = 10000, 160000, 50000, 200000, 2048, 64
_D, _DE, _ENC, _KMAX, _NOUT = 128, 16, 16, 2, 1
_EPS = 1e-05

_NSP = 50048
_NBLK = _NSP // 128
_ESP = 204800
_EB = _ESP // 128
_SINK = _NS
_NC = 4
_CW = 32
_SP = 2176
_SSINK = _S


def _mm_body(x_ref, w_ref, b_ref, o_ref, *, act):
    y = jnp.dot(x_ref[...], w_ref[...], preferred_element_type=jnp.float32)
    y = y + b_ref[...]
    if act == "relu":
        y = jnp.maximum(y, 0.0)
    o_ref[...] = y


def _tc_mm(x, w, b=None, act=None, block=400):
    m, k = x.shape
    n = w.shape[1]
    assert m % block == 0, (m, block)
    if b is None:
        b = jnp.zeros((n,), jnp.float32)
    b2 = b.reshape(1, n)
    return pl.pallas_call(
        functools.partial(_mm_body, act=act),
        grid=(m // block,),
        in_specs=[
            pl.BlockSpec((block, k), lambda i: (i, 0)),
            pl.BlockSpec((k, n), lambda i: (0, 0)),
            pl.BlockSpec((1, n), lambda i: (0, 0)),
        ],
        out_specs=pl.BlockSpec((block, n), lambda i: (i, 0)),
        out_shape=jax.ShapeDtypeStruct((m, n), jnp.float32),
    )(x, w, b2)


def _mm2_body(x_ref, w1_ref, b1_ref, w2_ref, o_ref):
    y = jnp.dot(x_ref[...], w1_ref[...], preferred_element_type=jnp.float32)
    y = jnp.maximum(y + b1_ref[...], 0.0)
    o_ref[...] = jnp.dot(y, w2_ref[...], preferred_element_type=jnp.float32)


def _tc_mm2(x, w1, b1, w2, block=800):
    m, k = x.shape
    n = w2.shape[1]
    assert m % block == 0
    return pl.pallas_call(
        _mm2_body,
        grid=(m // block,),
        in_specs=[
            pl.BlockSpec((block, k), lambda i: (i, 0)),
            pl.BlockSpec((k, w1.shape[1]), lambda i: (0, 0)),
            pl.BlockSpec((1, w1.shape[1]), lambda i: (0, 0)),
            pl.BlockSpec((w2.shape[0], n), lambda i: (0, 0)),
        ],
        out_specs=pl.BlockSpec((block, n), lambda i: (i, 0)),
        out_shape=jax.ShapeDtypeStruct((m, n), jnp.float32),
    )(x, w1, b1.reshape(1, -1), w2)


def _stats_body(x_ref, sum_ref, sq_ref):
    i = pl.program_id(0)

    @pl.when(i == 0)
    def _init():
        sum_ref[...] = jnp.zeros_like(sum_ref)
        sq_ref[...] = jnp.zeros_like(sq_ref)

    x = x_ref[...]
    sum_ref[...] += jnp.sum(x, axis=0, keepdims=True)
    sq_ref[...] += jnp.sum(x * x, axis=0, keepdims=True)


def _tc_colstats(x, rows, block=400):
    m, n = x.shape
    assert rows % block == 0
    return pl.pallas_call(
        _stats_body,
        grid=(rows // block,),
        in_specs=[pl.BlockSpec((block, n), lambda i: (i, 0))],
        out_specs=[pl.BlockSpec((1, n), lambda i: (0, 0)),
                   pl.BlockSpec((1, n), lambda i: (0, 0))],
        out_shape=[jax.ShapeDtypeStruct((1, n), jnp.float32),
                   jax.ShapeDtypeStruct((1, n), jnp.float32)],
    )(x)


def _norm_res_body(y_ref, m_ref, v_ref, prev_ref, o_ref):
    y = y_ref[...]
    hh = (y - m_ref[...]) * lax.rsqrt(v_ref[...] + _EPS)
    o_ref[...] = jnp.maximum(hh, 0.0) + prev_ref[...]


def _tc_norm_residual(y, mean, var, prev, block=1088):
    m, n = y.shape
    assert m % block == 0
    return pl.pallas_call(
        _norm_res_body,
        grid=(m // block,),
        in_specs=[
            pl.BlockSpec((block, n), lambda i: (i, 0)),
            pl.BlockSpec((1, n), lambda i: (0, 0)),
            pl.BlockSpec((1, n), lambda i: (0, 0)),
            pl.BlockSpec((block, n), lambda i: (i, 0)),
        ],
        out_specs=pl.BlockSpec((block, n), lambda i: (i, 0)),
        out_shape=jax.ShapeDtypeStruct((m, n), jnp.float32),
    )(y, mean, var, prev)


def _tail_body(parts_ref, ohk_ref, ohc_ref, kbf_ref, ksf_ref,
               kse_ref, cpe_ref, gk_ref, gc_ref,
               wsub_ref, bsub_ref, wbip_ref, bbip_ref,
               wd0a_ref, wd0b_ref, bd0_ref, wd1a_ref, wd1b_ref, bd1_ref,
               wd2_ref, bd2_ref, o_ref, mi_ref):
    f32 = jnp.float32
    subg = (parts_ref[0] + parts_ref[1])[:_S]
    ohk = ohk_ref[...]
    ohc = ohc_ref[...]
    kbf = kbf_ref[...]
    dot = functools.partial(jnp.dot, preferred_element_type=f32)
    subg = subg * ksf_ref[...] + dot(ohk, kse_ref[...]) + dot(ohc, cpe_ref[...])
    subg = jnp.maximum(dot(subg, wsub_ref[...]) + bsub_ref[...], 0.0)
    subg = jnp.maximum(dot(subg, wbip_ref[...]) + bbip_ref[...], 0.0)
    gate = jax.nn.sigmoid(dot(ohk, gk_ref[...]) + dot(ohc, gc_ref[...]))

    def seg_max(g, _):
        mask = kbf == g.astype(f32)
        m = jnp.max(jnp.where(mask, subg, -jnp.inf), axis=0, keepdims=True)
        mi_ref[pl.ds(g, 1), :] = m
        return 0
    lax.fori_loop(0, _G * _KMAX, seg_max, 0)
    mi = mi_ref[...]
    mi = jnp.where(mi == -jnp.inf, 0.0, mi)

    col = lax.broadcasted_iota(jnp.int32, (_S, _G * _KMAX), 1).astype(f32)
    ohkb = jnp.where(kbf == col, 1.0, 0.0)
    si = lax.dot_general(ohkb, subg * gate, (((0,), (0,)), ((), ())),
                         preferred_element_type=f32)

    xg = jnp.maximum(dot(mi, wd0a_ref[...]) + dot(si, wd0b_ref[...])
                     + bd0_ref[...], 0.0)
    xg3 = xg.reshape(_G, _KMAX, _D)
    y1 = jnp.maximum(dot(xg3[:, 0, :], wd1a_ref[...])
                     + dot(xg3[:, 1, :], wd1b_ref[...]) + bd1_ref[...], 0.0)
    o_ref[...] = dot(y1, wd2_ref[...]) + bd2_ref[...]


def _tc_tail(parts, ohk, ohc, kbf, ksf, ks_emb, comp_emb, gk, gc,
             W_sub, b_sub, W_bip, b_bip, wd0a, wd0b, b_dec0,
             wd1a, wd1b, b_d1, W_d2, b_d2):
    args = (parts, ohk, ohc, kbf, ksf, ks_emb, comp_emb, gk, gc,
            W_sub, b_sub.reshape(1, _D), W_bip, b_bip.reshape(1, _D),
            wd0a, wd0b, b_dec0.reshape(1, _D), wd1a, wd1b,
            b_d1.reshape(1, _D), W_d2, b_d2.reshape(1, _NOUT))
    return pl.pallas_call(
        _tail_body,
        grid=(1,),
        in_specs=[pl.BlockSpec(a.shape, lambda i, n=len(a.shape): (0,) * n)
                  for a in args],
        out_specs=pl.BlockSpec((_G, _NOUT), lambda i: (0, 0)),
        out_shape=jax.ShapeDtypeStruct((_G, _NOUT), jnp.float32),
        scratch_shapes=[pltpu.VMEM((_G * _KMAX, _D), jnp.float32)],
    )(*args)


_MESH = plsc.VectorSubcoreMesh(core_axis_name="c", subcore_axis_name="s")
_SC_PARAMS = pltpu.CompilerParams(use_tc_tiling_on_sc=False)


def _edge_body(pt, eat, src4, em4, dst2, zr, agg,
               sv, ev, dv, p_v, e_v, m_v, acc,
               g0, g1, s0, s1, i0, i1):
    core = lax.axis_index("c")
    sub = lax.axis_index("s")
    sem_g = (g0, g1)
    sem_s = (s0, s1)
    sem_i = (i0, i1)

    def issue_gather(bank, k, b):
        pltpu.async_copy(pt.at[sv.at[bank, k]], p_v.at[b], sem_g[b])
        pltpu.async_copy(eat.at[ev.at[bank, k]], e_v.at[b], sem_g[b])

    def wait_gather(bank, k, b):
        pltpu.make_async_copy(pt.at[sv.at[bank, k]], p_v.at[b], sem_g[b]).wait()
        pltpu.make_async_copy(eat.at[ev.at[bank, k]], e_v.at[b], sem_g[b]).wait()

    def issue_scatter(bank, k, b):
        pltpu.async_copy(m_v.at[b], acc.at[dv.at[bank, k]], sem_s[b], add=True)

    def wait_scatter(b):
        pltpu.make_async_copy(m_v.at[b], acc.at[dv.at[0, 0]], sem_s[b]).wait()

    def load_idx_sync(t, bank, base):
        pltpu.sync_copy(src4.at[pl.ds(base + 5 * t, 5)], sv.at[bank])
        pltpu.sync_copy(em4.at[pl.ds(base + 5 * t, 5)], ev.at[bank])
        pltpu.sync_copy(dst2.at[pl.ds(sub * 100 + 5 * t, 5)], dv.at[bank])

    def load_idx_async(t, bank, base):
        pltpu.async_copy(src4.at[pl.ds(base + 5 * t, 5)], sv.at[bank], sem_i[bank])
        pltpu.async_copy(em4.at[pl.ds(base + 5 * t, 5)], ev.at[bank], sem_i[bank])
        pltpu.async_copy(dst2.at[pl.ds(sub * 100 + 5 * t, 5)], dv.at[bank], sem_i[bank])

    def wait_idx(bank, base):
        pltpu.make_async_copy(src4.at[pl.ds(base, 5)], sv.at[bank], sem_i[bank]).wait()
        pltpu.make_async_copy(em4.at[pl.ds(base, 5)], ev.at[bank], sem_i[bank]).wait()
        pltpu.make_async_copy(dst2.at[pl.ds(base, 5)], dv.at[bank], sem_i[bank]).wait()

    def compute(b):
        def comp_iter(i, _):
            for rr in range(4):
                r = i * 4 + rr
                for hh in (0, 16):
                    a = p_v[b, r, pl.ds(hh, 16)]
                    bb = e_v[b, r, pl.ds(hh, 16)]
                    m_v[b, r, pl.ds(hh, 16)] = jnp.maximum(a + bb, 0.0)
            return 0
        lax.fori_loop(0, 32, comp_iter, 0)

    for cc in range(2):
        c = core * 2 + cc
        base = c * 1600 + sub * 100
        pltpu.sync_copy(zr, acc.at[pl.ds(sub * 3128, 3128)])
        plsc.subcore_barrier()

        load_idx_sync(0, 0, base)
        issue_gather(0, 0, 0)

        def period(t, tp):
            bt = tp
            nb = 1 - tp

            @pl.when(t > 0)
            def _():
                wait_scatter(tp)
                wait_scatter(1 - tp)

            @pl.when(t < 19)
            def _():
                load_idx_async(t + 1, nb, base)

            for k in range(5):
                pb = (tp + k) % 2
                if k < 4:
                    issue_gather(bt, k + 1, 1 - pb)
                else:
                    @pl.when(t < 19)
                    def _():
                        wait_idx(nb, base)
                        issue_gather(nb, 0, 1 - pb)
                wait_gather(bt, k, pb)
                if k >= 2:
                    wait_scatter(pb)
                compute(pb)
                issue_scatter(bt, k, pb)

        def two(s, _):
            period(2 * s, 0)
            period(2 * s + 1, 1)
            return 0

        lax.fori_loop(0, 10, two, 0)
        wait_scatter(0)
        wait_scatter(1)
        plsc.subcore_barrier()
        pltpu.sync_copy(acc.at[pl.ds(sub * 3128, 3128)],
                        agg.at[pl.ds(sub * 3128, 3128), c])
        plsc.subcore_barrier()


def _sc_edge_pass(pt_flat, eat_flat, src4, em4, dst2):
    zr = jnp.zeros((3128, _CW), jnp.float32)
    fn = pl.kernel(
        _edge_body,
        out_type=jax.ShapeDtypeStruct((_NSP, _NC, _CW), jnp.float32),
        mesh=_MESH,
        compiler_params=_SC_PARAMS,
        scratch_types=[
            pltpu.VMEM((2, 5, 128), jnp.int32),
            pltpu.VMEM((2, 5, 128), jnp.int32),
            pltpu.VMEM((2, 5, 128), jnp.int32),
            pltpu.VMEM((2, 128, _CW), jnp.float32),
            pltpu.VMEM((2, 128, _CW), jnp.float32),
            pltpu.VMEM((2, 128, _CW), jnp.float32),
            pltpu.VMEM_SHARED((_NSP, _CW), jnp.float32),
            pltpu.SemaphoreType.DMA,
            pltpu.SemaphoreType.DMA,
            pltpu.SemaphoreType.DMA,
            pltpu.SemaphoreType.DMA,
            pltpu.SemaphoreType.DMA,
            pltpu.SemaphoreType.DMA,
        ],
    )
    return fn(pt_flat, eat_flat, src4, em4, dst2, zr)


def _gather_body(h, idx2, out, iv, rows_v, s0, s1):
    core = lax.axis_index("c")
    sub = lax.axis_index("s")
    wid = sub * 2 + core
    sems = (s0, s1)

    def issue(j, b):
        pltpu.sync_copy(idx2.at[j * 32 + wid], iv.at[b])
        pltpu.async_copy(h.at[iv.at[b]], rows_v.at[b], sems[b])

    issue(0, 0)
    for j in range(13):
        b = j % 2
        if j + 1 < 13:
            @pl.when((j + 1) * 32 + wid < _NBLK)
            def _():
                issue(j + 1, 1 - b)

        @pl.when(j * 32 + wid < _NBLK)
        def _():
            pltpu.make_async_copy(h.at[iv.at[b]], rows_v.at[b], sems[b]).wait()
            pltpu.sync_copy(rows_v.at[b],
                            out.at[pl.ds((j * 32 + wid) * 128, 128)])


def _sc_gather(h, idx2):
    fn = pl.kernel(
        _gather_body,
        out_type=jax.ShapeDtypeStruct((_NSP, _D), jnp.float32),
        mesh=_MESH,
        compiler_params=_SC_PARAMS,
        scratch_types=[
            pltpu.VMEM((2, 128), jnp.int32),
            pltpu.VMEM((2, 128, _D), jnp.float32),
            pltpu.SemaphoreType.DMA,
            pltpu.SemaphoreType.DMA,
        ],
    )
    return fn(h, idx2)


def _segsum_body(xin, idx2, zr, out, iv, x_v, acc, sem):
    core = lax.axis_index("c")
    sub = lax.axis_index("s")
    pltpu.sync_copy(zr, acc.at[pl.ds(sub * 136, 136)])
    plsc.subcore_barrier()

    def step(j, _):
        off = j * 16 + sub

        @pl.when(off < 196 - core)
        def _():
            blk = core * 196 + off
            pltpu.sync_copy(idx2.at[pl.ds(blk, 1)], iv)
            pltpu.sync_copy(xin.at[pl.ds(blk * 128, 128)], x_v)
            pltpu.sync_copy(x_v, acc.at[iv.at[0]], add=True)
        return 0

    lax.fori_loop(0, 13, step, 0)
    plsc.subcore_barrier()
    pltpu.sync_copy(acc.at[pl.ds(sub * 136, 136)],
                    out.at[core, pl.ds(sub * 136, 136)])


def _sc_segsum(xin, idx2):
    zr = jnp.zeros((136, _D), jnp.float32)
    fn = pl.kernel(
        _segsum_body,
        out_type=jax.ShapeDtypeStruct((2, _SP, _D), jnp.float32),
        mesh=_MESH,
        compiler_params=_SC_PARAMS,
        scratch_types=[
            pltpu.VMEM((1, 128), jnp.int32),
            pltpu.VMEM((128, _D), jnp.float32),
            pltpu.VMEM_SHARED((_SP, _D), jnp.float32),
            pltpu.SemaphoreType.DMA,
        ],
    )
    return fn(xin, idx2, zr)


def kernel(x, edge_attr, W_in, b_in, W_edge, b_edge, W_msg1, W_self1,
           W_msg2, W_self2, W_oe, b_oe, ks_emb, comp_emb, W_sub, b_sub,
           W_bip, b_bip, ks_emb16, comp_emb16, W_gate, W_dec0, b_dec0,
           W_d1, b_d1, W_d2, b_d2, edge_index, subgraphs_nodes_mapper,
           combined_subgraphs, subgraphs_edges_mapper, subgraphs_batch,
           graph_id, ks, num_components):
    src = combined_subgraphs[0]
    dst = combined_subgraphs[1]

    pad_e = _ESP - _ES
    srcp = jnp.concatenate([src, jnp.zeros((pad_e,), jnp.int32)])
    emp = jnp.concatenate([subgraphs_edges_mapper, jnp.zeros((pad_e,), jnp.int32)])
    dstp = jnp.concatenate([dst, jnp.full((pad_e,), _SINK, jnp.int32)])
    offs = jnp.arange(_NC, dtype=jnp.int32)[:, None]
    src4 = (srcp[None, :] * _NC + offs).reshape(_NC * _EB, 128)
    em4 = (emp[None, :] * _NC + offs).reshape(_NC * _EB, 128)
    dst2 = dstp.reshape(_EB, 128)

    mapper_p = jnp.concatenate(
        [subgraphs_nodes_mapper, jnp.zeros((_NSP - _NS,), jnp.int32)]).reshape(_NBLK, 128)
    batch_p = jnp.concatenate(
        [subgraphs_batch, jnp.full((_NSP - _NS,), _SSINK, jnp.int32)]).reshape(_NBLK, 128)

    h = _tc_mm(x, W_in, b_in, act="relu", block=400)

    prev = _sc_gather(h, mapper_p)

    for Wm, Ws in ((W_msg1, W_self1), (W_msg2, W_self2)):
        p_t = _tc_mm(prev, Wm, block=1088).reshape(_NSP * _NC, _CW)
        ea_t = _tc_mm2(edge_attr, W_edge, b_edge, Wm, block=800).reshape(_E * _NC, _CW)
        agg = _sc_edge_pass(p_t, ea_t, src4, em4, dst2).reshape(_NSP, _D)
        y = _tc_mm(agg, Ws, block=1088)
        s, sq = _tc_colstats(y, _NS, block=400)
        mean = s / _NS
        var = sq / _NS - mean * mean
        prev = _tc_norm_residual(y, mean, var, prev, block=1088)

    hsub = _tc_mm(prev, W_oe, b_oe, act="relu", block=1088)
    parts = _sc_segsum(hsub, batch_p)

    lanes = jnp.arange(20, dtype=jnp.int32)
    ohk = (ks[:, None] == lanes).astype(jnp.float32)
    ohc = (num_components[:, None] == lanes).astype(jnp.float32)
    kbf = (ks + graph_id * _KMAX).astype(jnp.float32)[:, None]
    ksf = 1.0 / (ks + 1).astype(jnp.float32)[:, None]
    gk = ks_emb16 @ W_gate[:_ENC]
    gc = comp_emb16 @ W_gate[_ENC:]
    wd0a, wd0b = W_dec0[:_D], W_dec0[_D:]
    wd1a, wd1b = W_d1[:_D], W_d1[_D:]
    return _tc_tail(parts, ohk, ohc, kbf, ksf, ks_emb, comp_emb, gk, gc,
                    W_sub, b_sub, W_bip, b_bip, wd0a, wd0b, b_dec0,
                    wd1a, wd1b, b_d1, W_d2, b_d2)
